# baseline (speedup 1.0000x reference)
import functools

import jax
import jax.numpy as jnp
from jax import lax
from jax.experimental import pallas as pl
from jax.experimental.pallas import tpu as pltpu
from jax.experimental.pallas import tpu_sc as plsc

N = 10000
E = 320000
D = 128
B = 4096
L = 3
NC = 2
NS = 16
NW = NC * NS
CHUNK = 128
NCH = 80
E_PAD = NW * NCH * CHUNK
RPT = 640
N_ACC = NS * RPT
NBUF = 2

_MESH = plsc.VectorSubcoreMesh(core_axis_name="c", subcore_axis_name="s")


def _make_edge_pass(compute_deg):
  out_type = [jax.ShapeDtypeStruct((NC, N_ACC, D), jnp.float32)]
  if compute_deg:
    out_type.append(jax.ShapeDtypeStruct((NC * N_ACC,), jnp.float32))
  scratch = [
      pltpu.VMEM((NBUF, CHUNK), jnp.int32),
      pltpu.VMEM((NCH, CHUNK), jnp.int32),
      pltpu.VMEM((NBUF, CHUNK, D), jnp.float32),
      pltpu.VMEM((CHUNK,), jnp.float32),
      pltpu.VMEM((RPT,), jnp.float32),
      pltpu.VMEM_SHARED((N_ACC, D), jnp.float32),
      pltpu.VMEM_SHARED((N_ACC,), jnp.float32),
  ] + [pltpu.SemaphoreType.DMA] * (3 * NBUF + 1)

  @functools.partial(
      pl.kernel,
      out_type=tuple(out_type) if compute_deg else out_type[0],
      mesh=_MESH,
      scratch_types=scratch,
  )
  def edge_pass(src_flat, dst3, x, zrows, *rest):
    if compute_deg:
      (acc_out, deg_out, sidx_v, dst_v, rows_v, ones_v, deg_v, acc_sh,
       deg_sh, *sems) = rest
    else:
      (acc_out, sidx_v, dst_v, rows_v, ones_v, deg_v, acc_sh,
       deg_sh, *sems) = rest
    gsems, isems = sems[:NBUF], sems[NBUF:2 * NBUF]
    ssems, dsem = sems[2 * NBUF:3 * NBUF], sems[3 * NBUF]
    cid = lax.axis_index("c")
    sid = lax.axis_index("s")
    wid = cid * NS + sid

    pltpu.sync_copy(zrows, acc_sh.at[pl.ds(sid * RPT, RPT)])
    if compute_deg:
      for j in range(RPT // 16):
        deg_v[pl.ds(j * 16, 16)] = jnp.zeros((16,), jnp.float32)
      pltpu.sync_copy(deg_v, deg_sh.at[pl.ds(sid * RPT, RPT)])
      for j in range(CHUNK // 16):
        ones_v[pl.ds(j * 16, 16)] = jnp.ones((16,), jnp.float32)
    pltpu.sync_copy(dst3.at[wid], dst_v)
    plsc.subcore_barrier()

    def src_chunk(i):
      return src_flat.at[pl.ds((wid * NCH + i) * CHUNK, CHUNK)]

    for b in range(NBUF):
      pltpu.sync_copy(src_chunk(b), sidx_v.at[b])
      pltpu.async_copy(x.at[sidx_v.at[b]], rows_v.at[b], gsems[b])

    def body(g, carry):
      for b in range(NBUF):
        i = g * NBUF + b
        nxt = i + NBUF
        pltpu.make_async_copy(x.at[sidx_v.at[b]], rows_v.at[b],
                              gsems[b]).wait()

        @pl.when(nxt < NCH)
        def _():
          pltpu.async_copy(src_chunk(nxt), sidx_v.at[b], isems[b])

        pltpu.async_copy(rows_v.at[b], acc_sh.at[dst_v.at[i]], ssems[b],
                         add=True)
        if compute_deg:
          pltpu.async_copy(ones_v, deg_sh.at[dst_v.at[i]], dsem, add=True)

        @pl.when(nxt < NCH)
        def _():
          pltpu.make_async_copy(src_chunk(nxt), sidx_v.at[b],
                                isems[b]).wait()
          pltpu.make_async_copy(rows_v.at[b], acc_sh.at[dst_v.at[i]],
                                ssems[b]).wait()
          pltpu.async_copy(x.at[sidx_v.at[b]], rows_v.at[b], gsems[b])
      return carry

    lax.fori_loop(0, NCH // NBUF, body, 0)
    for b in range(NBUF):
      pltpu.make_async_copy(rows_v.at[b],
                            acc_sh.at[dst_v.at[NCH - NBUF + b]],
                            ssems[b]).wait()
    if compute_deg:
      def drain(i, carry):
        pltpu.make_async_copy(ones_v, deg_sh.at[dst_v.at[i]], dsem).wait()
        return carry

      lax.fori_loop(0, NCH, drain, 0)
    plsc.subcore_barrier()

    pltpu.sync_copy(acc_sh.at[pl.ds(sid * RPT, RPT)],
                    acc_out.at[cid, pl.ds(sid * RPT, RPT)])
    if compute_deg:
      pltpu.sync_copy(deg_sh.at[pl.ds(sid * RPT, RPT)], deg_v)
      pltpu.sync_copy(deg_v, deg_out.at[pl.ds(cid * N_ACC + sid * RPT, RPT)])

  return edge_pass


_edge_pass_deg = _make_edge_pass(True)
_edge_pass_nodeg = _make_edge_pass(False)

_GPT = (B * L) // NW


@functools.partial(
    pl.kernel,
    out_type=jax.ShapeDtypeStruct((B * L, D), jnp.float32),
    mesh=_MESH,
    scratch_types=[
        pltpu.VMEM((_GPT // CHUNK, CHUNK), jnp.int32),
        pltpu.VMEM((CHUNK, D), jnp.float32),
        pltpu.SemaphoreType.DMA,
    ],
)
def _tuple_gather(idx3, h, out, idx_v, rows_v, sem):
  cid = lax.axis_index("c")
  sid = lax.axis_index("s")
  wid = cid * NS + sid
  pltpu.sync_copy(idx3.at[wid], idx_v)
  for j in range(_GPT // CHUNK):
    pltpu.async_copy(h.at[idx_v.at[j]], rows_v, sem).wait()
    pltpu.sync_copy(rows_v, out.at[pl.ds(wid * _GPT + j * CHUNK, CHUNK)])


def _layer_body(acc_ref, x_ref, deg_ref, w_ref, b_ref, out_ref, *, relu):
  deg = deg_ref[0] + deg_ref[1]
  r = 1.0 / (deg + 1.0)
  h = (acc_ref[0] + acc_ref[1] + x_ref[...]) * r
  y = jnp.dot(h, w_ref[...], preferred_element_type=jnp.float32) + b_ref[...]
  out_ref[...] = jnp.maximum(y, 0.0) if relu else y


def _sage_layer(acc, x, deg3, w, b, relu):
  R = 1000
  return pl.pallas_call(
      functools.partial(_layer_body, relu=relu),
      grid=(N // R,),
      in_specs=[
          pl.BlockSpec((NC, R, D), lambda i: (0, i, 0)),
          pl.BlockSpec((R, D), lambda i: (i, 0)),
          pl.BlockSpec((NC, R, 1), lambda i: (0, i, 0)),
          pl.BlockSpec((D, D), lambda i: (0, 0)),
          pl.BlockSpec((1, D), lambda i: (0, 0)),
      ],
      out_specs=pl.BlockSpec((R, D), lambda i: (i, 0)),
      out_shape=jax.ShapeDtypeStruct((N, D), jnp.float32),
  )(acc, x, deg3, w, b)


def _pred_body(g_ref, w1_ref, b1_ref, w2_ref, b2_ref, out_ref):
  t = jnp.dot(g_ref[...], w1_ref[...], preferred_element_type=jnp.float32)
  t = jnp.maximum(t + b1_ref[...], 0.0)
  out_ref[...] = (
      jnp.dot(t, w2_ref[...], preferred_element_type=jnp.float32) + b2_ref[...]
  )


def _predictor(g, wp1, bp1, wp2, bp2):
  R = 512
  return pl.pallas_call(
      _pred_body,
      grid=(B // R,),
      in_specs=[
          pl.BlockSpec((R, L * D), lambda i: (i, 0)),
          pl.BlockSpec((L * D, 32), lambda i: (0, 0)),
          pl.BlockSpec((1, 32), lambda i: (0, 0)),
          pl.BlockSpec((32, 1000), lambda i: (0, 0)),
          pl.BlockSpec((1, 1000), lambda i: (0, 0)),
      ],
      out_specs=pl.BlockSpec((R, 1000), lambda i: (i, 0)),
      out_shape=jax.ShapeDtypeStruct((B, 1000), jnp.float32),
  )(g, wp1, bp1, wp2, bp2)


def kernel(graph, node_features, train_pos_samples, W1, b1, W2, b2,
           Wp1, bp1, Wp2, bp2):
  src = graph[0].astype(jnp.int32)
  dst = graph[1].astype(jnp.int32)
  npad = E_PAD - E
  pad = jnp.arange(npad, dtype=jnp.int32)
  src_p = jnp.concatenate([src, pad % N])
  dst3 = jnp.concatenate([dst, N + pad % (N_ACC - N)]).reshape(NW, NCH, CHUNK)
  zrows = jnp.zeros((RPT, D), jnp.float32)

  x = node_features
  acc1, deg = _edge_pass_deg(src_p, dst3, x, zrows)
  deg3 = deg.reshape(NC, N_ACC, 1)
  h1 = _sage_layer(acc1, x, deg3, W1, b1.reshape(1, D), True)
  acc2 = _edge_pass_nodeg(src_p, dst3, h1, zrows)
  h2 = _sage_layer(acc2, h1, deg3, W2, b2.reshape(1, D), False)

  idx3 = train_pos_samples.astype(jnp.int32).reshape(NW, _GPT // CHUNK, CHUNK)
  g = _tuple_gather(idx3, h2).reshape(B, L * D)
  return _predictor(g, Wp1, bp1.reshape(1, 32), Wp2, bp2.reshape(1, 1000))


# speedup vs baseline: 1.1210x; 1.1210x over previous
import functools

import jax
import jax.numpy as jnp
from jax import lax
from jax.experimental import pallas as pl
from jax.experimental.pallas import tpu as pltpu
from jax.experimental.pallas import tpu_sc as plsc

N = 10000
E = 320000
D = 128
B = 4096
L = 3
NC = 2
NS = 16
NW = NC * NS
CHUNK = 128
NCH = 80
E_PAD = NW * NCH * CHUNK
RPT = 640
N_ACC = NS * RPT
NBUF = 2

_MESH = plsc.VectorSubcoreMesh(core_axis_name="c", subcore_axis_name="s")


def _make_edge_pass(compute_deg):
  out_type = [jax.ShapeDtypeStruct((NC, N_ACC, D), jnp.float32)]
  if compute_deg:
    out_type.append(jax.ShapeDtypeStruct((NC * N_ACC,), jnp.float32))
  scratch = [
      pltpu.VMEM((NBUF, CHUNK), jnp.int32),
      pltpu.VMEM((NCH, CHUNK), jnp.int32),
      pltpu.VMEM((NBUF, CHUNK, D), jnp.float32),
      pltpu.VMEM((CHUNK,), jnp.float32),
      pltpu.VMEM((RPT,), jnp.float32),
      pltpu.VMEM_SHARED((N_ACC, D), jnp.float32),
      pltpu.VMEM_SHARED((N_ACC,), jnp.float32),
  ] + [pltpu.SemaphoreType.DMA] * (3 * NBUF + 1)

  @functools.partial(
      pl.kernel,
      out_type=tuple(out_type) if compute_deg else out_type[0],
      mesh=_MESH,
      scratch_types=scratch,
  )
  def edge_pass(src_flat, dst3, x, zrows, *rest):
    if compute_deg:
      (acc_out, deg_out, sidx_v, dst_v, rows_v, ones_v, deg_v, acc_sh,
       deg_sh, *sems) = rest
    else:
      (acc_out, sidx_v, dst_v, rows_v, ones_v, deg_v, acc_sh,
       deg_sh, *sems) = rest
    gsems, isems = sems[:NBUF], sems[NBUF:2 * NBUF]
    ssems, dsem = sems[2 * NBUF:3 * NBUF], sems[3 * NBUF]
    cid = lax.axis_index("c")
    sid = lax.axis_index("s")
    wid = cid * NS + sid

    pltpu.sync_copy(zrows, acc_sh.at[pl.ds(sid * RPT, RPT)])
    if compute_deg:
      for j in range(RPT // 16):
        deg_v[pl.ds(j * 16, 16)] = jnp.zeros((16,), jnp.float32)
      pltpu.sync_copy(deg_v, deg_sh.at[pl.ds(sid * RPT, RPT)])
      for j in range(CHUNK // 16):
        ones_v[pl.ds(j * 16, 16)] = jnp.ones((16,), jnp.float32)
    pltpu.sync_copy(dst3.at[wid], dst_v)
    plsc.subcore_barrier()

    def src_chunk(i):
      return src_flat.at[pl.ds((wid * NCH + i) * CHUNK, CHUNK)]

    for b in range(NBUF):
      pltpu.sync_copy(src_chunk(b), sidx_v.at[b])
      pltpu.async_copy(x.at[sidx_v.at[b]], rows_v.at[b], gsems[b])

    def body(g, carry):
      for b in range(NBUF):
        i = g * NBUF + b
        nxt = i + NBUF
        pltpu.make_async_copy(x.at[sidx_v.at[b]], rows_v.at[b],
                              gsems[b]).wait()

        @pl.when(nxt < NCH)
        def _():
          pltpu.async_copy(src_chunk(nxt), sidx_v.at[b], isems[b])

        pltpu.async_copy(rows_v.at[b], acc_sh.at[dst_v.at[i]], ssems[b],
                         add=True)
        if compute_deg:
          pltpu.async_copy(ones_v, deg_sh.at[dst_v.at[i]], dsem, add=True)

        @pl.when(nxt < NCH)
        def _():
          pltpu.make_async_copy(src_chunk(nxt), sidx_v.at[b],
                                isems[b]).wait()
          pltpu.make_async_copy(rows_v.at[b], acc_sh.at[dst_v.at[i]],
                                ssems[b]).wait()
          pltpu.async_copy(x.at[sidx_v.at[b]], rows_v.at[b], gsems[b])
      return carry

    lax.fori_loop(0, NCH // NBUF, body, 0)
    for b in range(NBUF):
      pltpu.make_async_copy(rows_v.at[b],
                            acc_sh.at[dst_v.at[NCH - NBUF + b]],
                            ssems[b]).wait()
    if compute_deg:
      def drain(i, carry):
        pltpu.make_async_copy(ones_v, deg_sh.at[dst_v.at[i]], dsem).wait()
        return carry

      lax.fori_loop(0, NCH, drain, 0)
    plsc.subcore_barrier()

    pltpu.sync_copy(acc_sh.at[pl.ds(sid * RPT, RPT)],
                    acc_out.at[cid, pl.ds(sid * RPT, RPT)])
    if compute_deg:
      pltpu.sync_copy(deg_sh.at[pl.ds(sid * RPT, RPT)], deg_v)
      pltpu.sync_copy(deg_v, deg_out.at[pl.ds(cid * N_ACC + sid * RPT, RPT)])

  return edge_pass


_edge_pass_deg = _make_edge_pass(True)
_edge_pass_nodeg = _make_edge_pass(False)

_KPT = (B * L) // (NW * CHUNK)


@functools.partial(
    pl.kernel,
    out_type=jax.ShapeDtypeStruct((B, L * D), jnp.float32),
    mesh=_MESH,
    scratch_types=[
        pltpu.VMEM((CHUNK,), jnp.int32),
        pltpu.VMEM((CHUNK, D), jnp.float32),
        pltpu.SemaphoreType.DMA,
    ],
)
def _tuple_gather(idx_flat, h, out, idx_v, rows_v, sem):
  cid = lax.axis_index("c")
  sid = lax.axis_index("s")
  wid = cid * NS + sid
  for m in range(_KPT):
    k = wid * _KPT + m
    l = k // 32
    boff = pl.multiple_of((k % 32) * CHUNK, CHUNK)
    pltpu.sync_copy(idx_flat.at[pl.ds(pl.multiple_of(k * CHUNK, CHUNK),
                                      CHUNK)], idx_v)
    pltpu.async_copy(h.at[idx_v], rows_v, sem).wait()
    pltpu.sync_copy(rows_v,
                    out.at[pl.ds(boff, CHUNK),
                           pl.ds(pl.multiple_of(l * D, D), D)])


_LR = 1024


def _layer_body(acc_ref, x_ref, deg_ref, w_ref, b_ref, eye_ref, out_ref, *,
                relu):
  d = deg_ref[0] + deg_ref[1]
  r = 1.0 / (d + 1.0)
  cols = [
      lax.dot_general(eye_ref[...], r[j:j + 1, :], (((1,), (1,)), ((), ())),
                      preferred_element_type=jnp.float32)
      for j in range(_LR // D)
  ]
  rcol = jnp.concatenate(cols, axis=0)
  h = (acc_ref[0] + acc_ref[1] + x_ref[...]) * rcol
  y = jnp.dot(h, w_ref[...], preferred_element_type=jnp.float32) + b_ref[...]
  out_ref[...] = jnp.maximum(y, 0.0) if relu else y


def _sage_layer(acc, x, degp, w, b, eye, relu):
  R = _LR
  return pl.pallas_call(
      functools.partial(_layer_body, relu=relu),
      grid=(N_ACC // R,),
      in_specs=[
          pl.BlockSpec((NC, R, D), lambda i: (0, i, 0)),
          pl.BlockSpec((R, D), lambda i: (i, 0)),
          pl.BlockSpec((NC, R // D, D), lambda i: (0, i, 0)),
          pl.BlockSpec((D, D), lambda i: (0, 0)),
          pl.BlockSpec((1, D), lambda i: (0, 0)),
          pl.BlockSpec((D, D), lambda i: (0, 0)),
      ],
      out_specs=pl.BlockSpec((R, D), lambda i: (i, 0)),
      out_shape=jax.ShapeDtypeStruct((N, D), jnp.float32),
  )(acc, x, degp, w, b, eye)


def _pred_body(g_ref, w1_ref, b1_ref, w2t_ref, b2c_ref, out_ref):
  t = jnp.dot(g_ref[...], w1_ref[...], preferred_element_type=jnp.float32)
  t = jnp.maximum(t + b1_ref[...], 0.0)
  out_ref[...] = lax.dot_general(
      w2t_ref[...], t, (((1,), (1,)), ((), ())),
      preferred_element_type=jnp.float32) + b2c_ref[...]


def _predictor(g, wp1, bp1, wp2, bp2):
  R = 512
  out_t = pl.pallas_call(
      _pred_body,
      grid=(B // R,),
      in_specs=[
          pl.BlockSpec((R, L * D), lambda i: (i, 0)),
          pl.BlockSpec((L * D, 32), lambda i: (0, 0)),
          pl.BlockSpec((1, 32), lambda i: (0, 0)),
          pl.BlockSpec((1000, 32), lambda i: (0, 0)),
          pl.BlockSpec((1000, 1), lambda i: (0, 0)),
      ],
      out_specs=pl.BlockSpec((1000, R), lambda i: (0, i)),
      out_shape=jax.ShapeDtypeStruct((1000, B), jnp.float32),
  )(g, wp1, bp1, wp2.T, bp2.reshape(1000, 1))
  return out_t.T


def kernel(graph, node_features, train_pos_samples, W1, b1, W2, b2,
           Wp1, bp1, Wp2, bp2):
  src = graph[0].astype(jnp.int32)
  dst = graph[1].astype(jnp.int32)
  npad = E_PAD - E
  pad = jnp.arange(npad, dtype=jnp.int32)
  src_p = jnp.concatenate([src, pad % N])
  dst3 = jnp.concatenate([dst, N + pad % (N_ACC - N)]).reshape(NW, NCH, CHUNK)
  zrows = jnp.zeros((RPT, D), jnp.float32)

  x = node_features
  acc1, deg = _edge_pass_deg(src_p, dst3, x, zrows)
  degp = deg.reshape(NC, N_ACC // D, D)
  eye = jnp.eye(D, dtype=jnp.float32)
  h1 = _sage_layer(acc1, x, degp, W1, b1.reshape(1, D), eye, True)
  acc2 = _edge_pass_nodeg(src_p, dst3, h1, zrows)
  h2 = _sage_layer(acc2, h1, degp, W2, b2.reshape(1, D), eye, False)

  idx_flat = train_pos_samples.astype(jnp.int32).T.reshape(B * L)
  g = _tuple_gather(idx_flat, h2)
  return _predictor(g, Wp1, bp1.reshape(1, 32), Wp2, bp2)


# speedup vs baseline: 1.2052x; 1.0751x over previous
import functools

import jax
import jax.numpy as jnp
from jax import lax
from jax.experimental import pallas as pl
from jax.experimental.pallas import tpu as pltpu
from jax.experimental.pallas import tpu_sc as plsc

N = 10000
E = 320000
D = 128
B = 4096
L = 3
NC = 2
NS = 16
NW = NC * NS
CHUNK = 128
NCHT = E // CHUNK
NCH = 78
NRING = 3
RPT = 640
N_ACC = NS * RPT
NBUF = 2

_MESH = plsc.VectorSubcoreMesh(core_axis_name="c", subcore_axis_name="s")


def _make_edge_pass(compute_deg):
  out_type = [jax.ShapeDtypeStruct((NC, N_ACC, D), jnp.float32)]
  if compute_deg:
    out_type.append(jax.ShapeDtypeStruct((NC * N_ACC,), jnp.float32))
  scratch = [
      pltpu.VMEM((NRING, 2, CHUNK), jnp.int32),
      pltpu.VMEM((NBUF, CHUNK, D), jnp.float32),
      pltpu.VMEM((CHUNK,), jnp.float32),
      pltpu.VMEM((RPT,), jnp.float32),
      pltpu.VMEM_SHARED((N_ACC, D), jnp.float32),
      pltpu.VMEM_SHARED((N_ACC,), jnp.float32),
  ] + [pltpu.SemaphoreType.DMA] * (2 * NBUF + 2 * NRING)

  @functools.partial(
      pl.kernel,
      out_type=tuple(out_type) if compute_deg else out_type[0],
      mesh=_MESH,
      scratch_types=scratch,
  )
  def edge_pass(graph, x, zrows, *rest):
    if compute_deg:
      (acc_out, deg_out, pidx_v, rows_v, ones_v, deg_v, acc_sh,
       deg_sh, *sems) = rest
    else:
      (acc_out, pidx_v, rows_v, ones_v, deg_v, acc_sh,
       deg_sh, *sems) = rest
    gsems, ssems = sems[:NBUF], sems[NBUF:2 * NBUF]
    isems = sems[2 * NBUF:2 * NBUF + NRING]
    dsems = sems[2 * NBUF + NRING:]
    cid = lax.axis_index("c")
    sid = lax.axis_index("s")
    wid = cid * NS + sid

    pltpu.sync_copy(zrows, acc_sh.at[pl.ds(sid * RPT, RPT)])
    if compute_deg:
      for j in range(RPT // 16):
        deg_v[pl.ds(j * 16, 16)] = jnp.zeros((16,), jnp.float32)
      pltpu.sync_copy(deg_v, deg_sh.at[pl.ds(sid * RPT, RPT)])
      for j in range(CHUNK // 16):
        ones_v[pl.ds(j * 16, 16)] = jnp.ones((16,), jnp.float32)
    sc = (625 * wid) // 8
    cnt = (625 * (wid + 1)) // 8 - sc
    plsc.subcore_barrier()

    def stage(i, s):
      c = pl.multiple_of((sc + i) * CHUNK, CHUNK)
      return pltpu.make_async_copy(graph.at[:, pl.ds(c, CHUNK)],
                                   pidx_v.at[s], isems[s])

    def gather(s, rb):
      return pltpu.make_async_copy(x.at[pidx_v.at[s, 0]],
                                   rows_v.at[rb], gsems[rb])

    def scatter(s, rb):
      return pltpu.make_async_copy(rows_v.at[rb],
                                   acc_sh.at[pidx_v.at[s, 1]],
                                   ssems[rb])

    def deg_add(s):
      return pltpu.make_async_copy(ones_v, deg_sh.at[pidx_v.at[s, 1]],
                                   dsems[s])

    for s in range(NRING):
      stage(s, s).start()
    for b in range(NBUF):
      stage(b, b).wait()
      gather(b, b).start()
    if compute_deg:
      deg_add(0).start(add=True)

    def body(g, carry):
      for b in range(6):
        j = g * 6 + b
        rb = b % 2
        ib = b % 3
        gather(ib, rb).wait()
        scatter(ib, rb).start(add=True)
        if compute_deg:
          @pl.when(j + 1 < NCH)
          def _():
            deg_add((b + 1) % 3).start(add=True)

        @pl.when(j + 2 < NCH)
        def _():
          scatter(ib, rb).wait()
          stage(j + 2, (b + 2) % 3).wait()
          gather((b + 2) % 3, rb).start()

        @pl.when(j + 3 < NCH)
        def _():
          if compute_deg:
            deg_add(ib).wait()
          stage(j + 3, ib).start()
      return carry

    lax.fori_loop(0, NCH // 6, body, 0)
    for t in range(NBUF):
      j = NCH - NBUF + t
      scatter(j % 3, j % 2).wait()
    if compute_deg:
      for j in range(NCH - 3, NCH):
        deg_add(j % 3).wait()

    @pl.when(cnt > NCH)
    def _():
      stage(NCH, 0).start()
      stage(NCH, 0).wait()
      gather(0, 0).start()
      gather(0, 0).wait()
      pltpu.sync_copy(rows_v.at[0], acc_sh.at[pidx_v.at[0, 1]], add=True)
      if compute_deg:
        pltpu.sync_copy(ones_v, deg_sh.at[pidx_v.at[0, 1]], add=True)

    plsc.subcore_barrier()

    pltpu.sync_copy(acc_sh.at[pl.ds(sid * RPT, RPT)],
                    acc_out.at[cid, pl.ds(sid * RPT, RPT)])
    if compute_deg:
      pltpu.sync_copy(deg_sh.at[pl.ds(sid * RPT, RPT)], deg_v)
      pltpu.sync_copy(deg_v, deg_out.at[pl.ds(cid * N_ACC + sid * RPT, RPT)])

  return edge_pass


_edge_pass_deg = _make_edge_pass(True)
_edge_pass_nodeg = _make_edge_pass(False)

_KPT = (B * L) // (NW * CHUNK)


@functools.partial(
    pl.kernel,
    out_type=jax.ShapeDtypeStruct((B, L * D), jnp.float32),
    mesh=_MESH,
    scratch_types=[
        pltpu.VMEM((CHUNK,), jnp.int32),
        pltpu.VMEM((CHUNK, D), jnp.float32),
        pltpu.SemaphoreType.DMA,
    ],
)
def _tuple_gather(idx_flat, h, out, idx_v, rows_v, sem):
  cid = lax.axis_index("c")
  sid = lax.axis_index("s")
  wid = cid * NS + sid
  for m in range(_KPT):
    k = wid * _KPT + m
    l = k // 32
    boff = pl.multiple_of((k % 32) * CHUNK, CHUNK)
    pltpu.sync_copy(idx_flat.at[pl.ds(pl.multiple_of(k * CHUNK, CHUNK),
                                      CHUNK)], idx_v)
    pltpu.async_copy(h.at[idx_v], rows_v, sem).wait()
    pltpu.sync_copy(rows_v,
                    out.at[pl.ds(boff, CHUNK),
                           pl.ds(pl.multiple_of(l * D, D), D)])


_LR = 1024


def _layer_body(acc_ref, x_ref, deg_ref, w_ref, b_ref, eye_ref, out_ref, *,
                relu):
  d = deg_ref[0] + deg_ref[1]
  r = 1.0 / (d + 1.0)
  cols = [
      lax.dot_general(eye_ref[...], r[j:j + 1, :], (((1,), (1,)), ((), ())),
                      preferred_element_type=jnp.float32)
      for j in range(_LR // D)
  ]
  rcol = jnp.concatenate(cols, axis=0)
  h = (acc_ref[0] + acc_ref[1] + x_ref[...]) * rcol
  y = jnp.dot(h, w_ref[...], preferred_element_type=jnp.float32) + b_ref[...]
  out_ref[...] = jnp.maximum(y, 0.0) if relu else y


def _sage_layer(acc, x, degp, w, b, eye, relu):
  R = _LR
  return pl.pallas_call(
      functools.partial(_layer_body, relu=relu),
      grid=(N_ACC // R,),
      in_specs=[
          pl.BlockSpec((NC, R, D), lambda i: (0, i, 0)),
          pl.BlockSpec((R, D), lambda i: (i, 0)),
          pl.BlockSpec((NC, R // D, D), lambda i: (0, i, 0)),
          pl.BlockSpec((D, D), lambda i: (0, 0)),
          pl.BlockSpec((1, D), lambda i: (0, 0)),
          pl.BlockSpec((D, D), lambda i: (0, 0)),
      ],
      out_specs=pl.BlockSpec((R, D), lambda i: (i, 0)),
      out_shape=jax.ShapeDtypeStruct((N, D), jnp.float32),
  )(acc, x, degp, w, b, eye)


def _pred_body(g_ref, w1_ref, b1_ref, w2t_ref, b2c_ref, out_ref):
  t = jnp.dot(g_ref[...], w1_ref[...], preferred_element_type=jnp.float32)
  t = jnp.maximum(t + b1_ref[...], 0.0)
  out_ref[...] = lax.dot_general(
      w2t_ref[...], t, (((1,), (1,)), ((), ())),
      preferred_element_type=jnp.float32) + b2c_ref[...]


def _predictor(g, wp1, bp1, wp2, bp2):
  R = 512
  out_t = pl.pallas_call(
      _pred_body,
      grid=(B // R,),
      in_specs=[
          pl.BlockSpec((R, L * D), lambda i: (i, 0)),
          pl.BlockSpec((L * D, 32), lambda i: (0, 0)),
          pl.BlockSpec((1, 32), lambda i: (0, 0)),
          pl.BlockSpec((1000, 32), lambda i: (0, 0)),
          pl.BlockSpec((1000, 1), lambda i: (0, 0)),
      ],
      out_specs=pl.BlockSpec((1000, R), lambda i: (0, i)),
      out_shape=jax.ShapeDtypeStruct((1000, B), jnp.float32),
  )(g, wp1, bp1, wp2.T, bp2.reshape(1000, 1))
  return out_t.T


def kernel(graph, node_features, train_pos_samples, W1, b1, W2, b2,
           Wp1, bp1, Wp2, bp2):
  graph = graph.astype(jnp.int32)
  zrows = jnp.zeros((RPT, D), jnp.float32)

  x = node_features
  acc1, deg = _edge_pass_deg(graph, x, zrows)
  degp = deg.reshape(NC, N_ACC // D, D)
  eye = jnp.eye(D, dtype=jnp.float32)
  h1 = _sage_layer(acc1, x, degp, W1, b1.reshape(1, D), eye, True)
  acc2 = _edge_pass_nodeg(graph, h1, zrows)
  h2 = _sage_layer(acc2, h1, degp, W2, b2.reshape(1, D), eye, False)

  idx_flat = train_pos_samples.astype(jnp.int32).T.reshape(B * L)
  g = _tuple_gather(idx_flat, h2)
  return _predictor(g, Wp1, bp1.reshape(1, 32), Wp2, bp2)
